# Initial kernel scaffold; baseline (speedup 1.0000x reference)
import functools

import jax
import jax.numpy as jnp
from jax.experimental import pallas as pl

B, S, D, H, E, K, F = 1, 2048, 768, 12, 8, 2, 768
DH = D // H
NEG = -1e30


def _dot(a, b):
    return jnp.dot(a, b, preferred_element_type=jnp.float32)


def _rmsnorm(x, w, eps=1e-6):
    return x * jax.lax.rsqrt(jnp.mean(x * x, axis=-1, keepdims=True) + eps) * w


def _embed_qkv_body(c_ref, win_ref, bin_ref, ln1_ref, wq_ref, wk_ref, wv_ref,
                    x_ref, q_ref, k_ref, v_ref):
    x = c_ref[...] * win_ref[...] + bin_ref[...]
    x_ref[...] = x
    h = _rmsnorm(x, ln1_ref[...]).astype(jnp.bfloat16)
    q_ref[...] = _dot(h, wq_ref[...]).astype(jnp.bfloat16)
    k_ref[...] = _dot(h, wk_ref[...]).astype(jnp.bfloat16)
    v_ref[...] = _dot(h, wv_ref[...]).astype(jnp.bfloat16)


def _embed_qkv(c_col, W_in, b_in, ln1, Wq, Wk, Wv, bs=512):
    nb = S // bs
    return pl.pallas_call(
        _embed_qkv_body,
        grid=(nb,),
        in_specs=[
            pl.BlockSpec((bs, 1), lambda i: (i, 0)),
            pl.BlockSpec((1, D), lambda i: (0, 0)),
            pl.BlockSpec((1, D), lambda i: (0, 0)),
            pl.BlockSpec((1, D), lambda i: (0, 0)),
            pl.BlockSpec((D, D), lambda i: (0, 0)),
            pl.BlockSpec((D, D), lambda i: (0, 0)),
            pl.BlockSpec((D, D), lambda i: (0, 0)),
        ],
        out_specs=[
            pl.BlockSpec((bs, D), lambda i: (i, 0)),
            pl.BlockSpec((bs, D), lambda i: (i, 0)),
            pl.BlockSpec((bs, D), lambda i: (i, 0)),
            pl.BlockSpec((bs, D), lambda i: (i, 0)),
        ],
        out_shape=[
            jax.ShapeDtypeStruct((S, D), jnp.float32),
            jax.ShapeDtypeStruct((S, D), jnp.bfloat16),
            jax.ShapeDtypeStruct((S, D), jnp.bfloat16),
            jax.ShapeDtypeStruct((S, D), jnp.bfloat16),
        ],
    )(c_col, W_in, b_in.reshape(1, D), ln1.reshape(1, D), Wq, Wk, Wv)


def _flash_body(q_ref, k_ref, v_ref, o_ref, *, bq, bk):
    i = pl.program_id(1)
    q = q_ref[0] * jnp.bfloat16(1.0 / (DH ** 0.5))
    rows = i * bq + jax.lax.broadcasted_iota(jnp.int32, (bq, bk), 0)

    def step(j, carry):
        m, l, acc = carry
        kb = k_ref[0, pl.ds(j * bk, bk), :]
        vb = v_ref[0, pl.ds(j * bk, bk), :]
        s = jax.lax.dot_general(q, kb, (((1,), (1,)), ((), ())),
                                preferred_element_type=jnp.float32)
        cols = j * bk + jax.lax.broadcasted_iota(jnp.int32, (bq, bk), 1)
        s = jnp.where(cols <= rows, s, NEG)
        m_new = jnp.maximum(m, jnp.max(s, axis=-1, keepdims=True))
        p = jnp.exp(s - m_new)
        corr = jnp.exp(m - m_new)
        l = l * corr + jnp.sum(p, axis=-1, keepdims=True)
        acc = acc * corr + _dot(p.astype(jnp.bfloat16), vb)
        return m_new, l, acc

    m0 = jnp.full((bq, 1), NEG, jnp.float32)
    l0 = jnp.zeros((bq, 1), jnp.float32)
    a0 = jnp.zeros((bq, DH), jnp.float32)
    m, l, acc = jax.lax.fori_loop(0, i + 1, step, (m0, l0, a0))
    o_ref[0] = (acc / l).astype(jnp.bfloat16)


def _flash_attn(q, k, v, bq=512, bk=512):
    nq = S // bq
    body = functools.partial(_flash_body, bq=bq, bk=bk)
    return pl.pallas_call(
        body,
        grid=(H, nq),
        in_specs=[
            pl.BlockSpec((1, bq, DH), lambda h, i: (h, i, 0)),
            pl.BlockSpec((1, S, DH), lambda h, i: (h, 0, 0)),
            pl.BlockSpec((1, S, DH), lambda h, i: (h, 0, 0)),
        ],
        out_specs=pl.BlockSpec((1, bq, DH), lambda h, i: (h, i, 0)),
        out_shape=jax.ShapeDtypeStruct((H, S, DH), jnp.bfloat16),
    )(q, k, v)


def _post_router_body(x_ref, o_ref, wo_ref, ln2_ref, wr_ref,
                      x2_ref, h2_ref, g_ref, p_ref):
    x2 = x_ref[...] + _dot(o_ref[...], wo_ref[...])
    x2_ref[...] = x2
    h2 = _rmsnorm(x2, ln2_ref[...])
    h2_ref[...] = h2.astype(jnp.bfloat16)
    logits = _dot(h2, wr_ref[...])
    lane = jax.lax.broadcasted_iota(jnp.int32, logits.shape, 1)
    rl = jnp.where(lane < E, logits, NEG)
    mx = jnp.max(rl, axis=-1, keepdims=True)
    ex = jnp.exp(rl - mx)
    probs = ex / jnp.sum(ex, axis=-1, keepdims=True)
    p_ref[...] = probs
    v1 = jnp.max(probs, axis=-1, keepdims=True)
    i1 = jnp.min(jnp.where((probs == v1) & (lane < E), lane, 128),
                 axis=-1, keepdims=True)
    probs2 = jnp.where((lane == i1) | (lane >= E), NEG, probs)
    v2 = jnp.max(probs2, axis=-1, keepdims=True)
    i2 = jnp.min(jnp.where((probs2 == v2) & (lane < E), lane, 128),
                 axis=-1, keepdims=True)
    tot = v1 + v2
    gates = (jnp.where(lane == i1, v1 / tot, 0.0)
             + jnp.where(lane == i2, v2 / tot, 0.0))
    sg = jax.nn.sigmoid(logits[:, E:E + 1])
    g_ref[...] = gates + jnp.where(lane == E, sg, 0.0)


def _post_router(x, o, Wo, ln2, Wrcat, bs=512):
    nb = S // bs
    return pl.pallas_call(
        _post_router_body,
        grid=(nb,),
        in_specs=[
            pl.BlockSpec((bs, D), lambda i: (i, 0)),
            pl.BlockSpec((bs, D), lambda i: (i, 0)),
            pl.BlockSpec((D, D), lambda i: (0, 0)),
            pl.BlockSpec((1, D), lambda i: (0, 0)),
            pl.BlockSpec((D, 128), lambda i: (0, 0)),
        ],
        out_specs=[
            pl.BlockSpec((bs, D), lambda i: (i, 0)),
            pl.BlockSpec((bs, D), lambda i: (i, 0)),
            pl.BlockSpec((bs, 128), lambda i: (i, 0)),
            pl.BlockSpec((bs, 128), lambda i: (i, 0)),
        ],
        out_shape=[
            jax.ShapeDtypeStruct((S, D), jnp.float32),
            jax.ShapeDtypeStruct((S, D), jnp.bfloat16),
            jax.ShapeDtypeStruct((S, 128), jnp.float32),
            jax.ShapeDtypeStruct((S, 128), jnp.float32),
        ],
    )(x, o, Wo, ln2.reshape(1, D), Wrcat)


def _moe_body(h2_ref, w1_ref, w3_ref, w2_ref, g_ref, out_ref):
    e = pl.program_id(1)
    h2 = h2_ref[...]
    a = _dot(h2, w1_ref[0])
    bmat = _dot(h2, w3_ref[0])
    inner = (a * jax.nn.sigmoid(a)) * bmat
    ye = _dot(inner.astype(jnp.bfloat16), w2_ref[0])
    contrib = ye * g_ref[0, 0]

    @pl.when(e == 0)
    def _():
        out_ref[...] = contrib

    @pl.when(e != 0)
    def _():
        out_ref[...] += contrib


def _moe_dense(h2b, W1c, W3c, W2c, gcol, bs=512):
    nb = S // bs
    return pl.pallas_call(
        _moe_body,
        grid=(nb, E + 1),
        in_specs=[
            pl.BlockSpec((bs, D), lambda i, e: (i, 0)),
            pl.BlockSpec((1, D, F), lambda i, e: (e, 0, 0)),
            pl.BlockSpec((1, D, F), lambda i, e: (e, 0, 0)),
            pl.BlockSpec((1, F, D), lambda i, e: (e, 0, 0)),
            pl.BlockSpec((1, 1, bs, 1), lambda i, e: (e, i, 0, 0)),
        ],
        out_specs=pl.BlockSpec((bs, D), lambda i, e: (i, 0)),
        out_shape=jax.ShapeDtypeStruct((S, D), jnp.float32),
    )(h2b, W1c, W3c, W2c, gcol)


def _final_body(x2_ref, moe_ref, lnf_ref, wh_ref, bh_ref, t_ref, m_ref,
                g_ref, p_ref, acc_ref, loss_ref, *, nb):
    i = pl.program_id(0)

    @pl.when(i == 0)
    def _():
        acc_ref[...] = jnp.zeros_like(acc_ref)

    x3 = x2_ref[...] + moe_ref[...]
    hf = _rmsnorm(x3, lnf_ref[...])
    pred = _dot(hf, wh_ref[...])[:, :1] + bh_ref[0, 0]
    diff = pred - t_ref[...]
    msk = m_ref[...]
    lane = jax.lax.broadcasted_iota(jnp.int32, g_ref.shape, 1)
    fsel = ((g_ref[...] > 0) & (lane < E)).astype(jnp.float32)
    acc_ref[0:1, 0:1] += jnp.sum(diff * diff * msk).reshape(1, 1)
    acc_ref[1:2, 0:1] += jnp.sum(msk).reshape(1, 1)
    acc_ref[2:3, :] += jnp.sum(fsel, axis=0, keepdims=True)
    acc_ref[3:4, :] += jnp.sum(p_ref[...], axis=0, keepdims=True)

    @pl.when(i == nb - 1)
    def _():
        mse = acc_ref[0, 0] / jnp.maximum(acc_ref[1, 0], 1.0)
        lane1 = jax.lax.broadcasted_iota(jnp.int32, (1, 128), 1)
        fp = jnp.where(lane1 < E, acc_ref[2:3, :] * acc_ref[3:4, :], 0.0)
        aux = (E / (S * S * 1.0)) * jnp.sum(fp)
        loss_ref[0, 0] = mse + 0.02 * aux


def _final_loss(x2, moe, lnf, Whcat, b_head, t_col, m_col, gates, probs,
                bs=512):
    nb = S // bs
    body = functools.partial(_final_body, nb=nb)
    acc, loss = pl.pallas_call(
        body,
        grid=(nb,),
        in_specs=[
            pl.BlockSpec((bs, D), lambda i: (i, 0)),
            pl.BlockSpec((bs, D), lambda i: (i, 0)),
            pl.BlockSpec((1, D), lambda i: (0, 0)),
            pl.BlockSpec((D, 128), lambda i: (0, 0)),
            pl.BlockSpec((1, 1), lambda i: (0, 0)),
            pl.BlockSpec((bs, 1), lambda i: (i, 0)),
            pl.BlockSpec((bs, 1), lambda i: (i, 0)),
            pl.BlockSpec((bs, 128), lambda i: (i, 0)),
            pl.BlockSpec((bs, 128), lambda i: (i, 0)),
        ],
        out_specs=[
            pl.BlockSpec((4, 128), lambda i: (0, 0)),
            pl.BlockSpec((1, 1), lambda i: (0, 0)),
        ],
        out_shape=[
            jax.ShapeDtypeStruct((4, 128), jnp.float32),
            jax.ShapeDtypeStruct((1, 1), jnp.float32),
        ],
    )(x2, moe, lnf.reshape(1, D), Whcat, b_head.reshape(1, 1), t_col, m_col,
      gates, probs)
    return loss


def kernel(context, target, mask, W_in, b_in, ln1, ln2, lnf, Wq, Wk, Wv, Wo,
           W_router, W1, W3, W2, Ws1, Ws3, Ws2, W_sg, W_head, b_head):
    bf = jnp.bfloat16
    c_col = context.reshape(S, 1)
    x, q, k, v = _embed_qkv(c_col, W_in, b_in, ln1,
                            Wq.astype(bf), Wk.astype(bf), Wv.astype(bf))

    qh = q.reshape(S, H, DH).transpose(1, 0, 2)
    kh = k.reshape(S, H, DH).transpose(1, 0, 2)
    vh = v.reshape(S, H, DH).transpose(1, 0, 2)
    oh = _flash_attn(qh, kh, vh)
    o = oh.transpose(1, 0, 2).reshape(S, D)

    Wrcat = jnp.zeros((D, 128), jnp.float32)
    Wrcat = Wrcat.at[:, :E].set(W_router).at[:, E:E + 1].set(W_sg)
    x2, h2b, gates, probs = _post_router(x, o.astype(bf), Wo.astype(bf),
                                         ln2, Wrcat)

    W1c = jnp.concatenate([W1, Ws1[None]], axis=0).astype(bf)
    W3c = jnp.concatenate([W3, Ws3[None]], axis=0).astype(bf)
    W2c = jnp.concatenate([W2, Ws2[None]], axis=0).astype(bf)
    bs = 512
    gcol = gates[:, :E + 1].T.reshape(E + 1, S // bs, bs, 1)
    moe = _moe_dense(h2b, W1c, W3c, W2c, gcol, bs=bs)

    Whcat = jnp.zeros((D, 128), jnp.float32).at[:, :1].set(W_head)
    loss = _final_loss(x2, moe, lnf, Whcat, b_head, target.reshape(S, 1),
                       mask.reshape(S, 1), gates, probs)
    return jnp.reshape(loss, ())


# speedup vs baseline: 1.4587x; 1.4587x over previous
import functools

import jax
import jax.numpy as jnp
from jax.experimental import pallas as pl

B, S, D, H, E, K, F = 1, 2048, 768, 12, 8, 2, 768
DH = D // H
NEG = -1e30


def _dot(a, b):
    return jnp.dot(a, b, preferred_element_type=jnp.float32)


def _rmsnorm(x, w, eps=1e-6):
    return x * jax.lax.rsqrt(jnp.mean(x * x, axis=-1, keepdims=True) + eps) * w


def _embed_qkv_body(c_ref, win_ref, bin_ref, ln1_ref, wq_ref, wk_ref, wv_ref,
                    x_ref, q_ref, k_ref, v_ref):
    x = c_ref[...] * win_ref[...] + bin_ref[...]
    x_ref[...] = x
    h = _rmsnorm(x, ln1_ref[...]).astype(jnp.bfloat16)
    q_ref[...] = _dot(h, wq_ref[...]).astype(jnp.bfloat16)
    k_ref[...] = _dot(h, wk_ref[...]).astype(jnp.bfloat16)
    v_ref[...] = _dot(h, wv_ref[...]).astype(jnp.bfloat16)


def _embed_qkv(c_col, W_in, b_in, ln1, Wq, Wk, Wv, bs=512):
    nb = S // bs
    return pl.pallas_call(
        _embed_qkv_body,
        grid=(nb,),
        in_specs=[
            pl.BlockSpec((bs, 1), lambda i: (i, 0)),
            pl.BlockSpec((1, D), lambda i: (0, 0)),
            pl.BlockSpec((1, D), lambda i: (0, 0)),
            pl.BlockSpec((1, D), lambda i: (0, 0)),
            pl.BlockSpec((D, D), lambda i: (0, 0)),
            pl.BlockSpec((D, D), lambda i: (0, 0)),
            pl.BlockSpec((D, D), lambda i: (0, 0)),
        ],
        out_specs=[
            pl.BlockSpec((bs, D), lambda i: (i, 0)),
            pl.BlockSpec((bs, D), lambda i: (i, 0)),
            pl.BlockSpec((bs, D), lambda i: (i, 0)),
            pl.BlockSpec((bs, D), lambda i: (i, 0)),
        ],
        out_shape=[
            jax.ShapeDtypeStruct((S, D), jnp.float32),
            jax.ShapeDtypeStruct((S, D), jnp.bfloat16),
            jax.ShapeDtypeStruct((S, D), jnp.bfloat16),
            jax.ShapeDtypeStruct((S, D), jnp.bfloat16),
        ],
    )(c_col, W_in, b_in.reshape(1, D), ln1.reshape(1, D), Wq, Wk, Wv)


def _flash_body(q_ref, k_ref, v_ref, o_ref, *, bq, bk):
    i = pl.program_id(1)
    q = q_ref[0] * jnp.bfloat16(1.0 / (DH ** 0.5))
    rows = i * bq + jax.lax.broadcasted_iota(jnp.int32, (bq, bk), 0)

    def step(j, carry):
        m, l, acc = carry
        kb = k_ref[0, pl.ds(j * bk, bk), :]
        vb = v_ref[0, pl.ds(j * bk, bk), :]
        s = jax.lax.dot_general(q, kb, (((1,), (1,)), ((), ())),
                                preferred_element_type=jnp.float32)
        cols = j * bk + jax.lax.broadcasted_iota(jnp.int32, (bq, bk), 1)
        s = jnp.where(cols <= rows, s, NEG)
        m_new = jnp.maximum(m, jnp.max(s, axis=-1, keepdims=True))
        p = jnp.exp(s - m_new)
        corr = jnp.exp(m - m_new)
        l = l * corr + jnp.sum(p, axis=-1, keepdims=True)
        acc = acc * corr + _dot(p.astype(jnp.bfloat16), vb)
        return m_new, l, acc

    m0 = jnp.full((bq, 1), NEG, jnp.float32)
    l0 = jnp.zeros((bq, 1), jnp.float32)
    a0 = jnp.zeros((bq, DH), jnp.float32)
    m, l, acc = jax.lax.fori_loop(0, i + 1, step, (m0, l0, a0))
    o_ref[0] = (acc / l).astype(jnp.bfloat16)


def _flash_attn(q, k, v, bq=512, bk=512):
    nq = S // bq
    body = functools.partial(_flash_body, bq=bq, bk=bk)
    return pl.pallas_call(
        body,
        grid=(H, nq),
        in_specs=[
            pl.BlockSpec((1, bq, DH), lambda h, i: (h, i, 0)),
            pl.BlockSpec((1, S, DH), lambda h, i: (h, 0, 0)),
            pl.BlockSpec((1, S, DH), lambda h, i: (h, 0, 0)),
        ],
        out_specs=pl.BlockSpec((1, bq, DH), lambda h, i: (h, i, 0)),
        out_shape=jax.ShapeDtypeStruct((H, S, DH), jnp.bfloat16),
    )(q, k, v)


def _post_router_body(x_ref, o_ref, wo_ref, ln2_ref, wr_ref,
                      x2_ref, h2_ref, g_ref, p_ref):
    x2 = x_ref[...] + _dot(o_ref[...], wo_ref[...])
    x2_ref[...] = x2
    h2 = _rmsnorm(x2, ln2_ref[...])
    h2_ref[...] = h2.astype(jnp.bfloat16)
    logits = _dot(h2, wr_ref[...])
    lane = jax.lax.broadcasted_iota(jnp.int32, logits.shape, 1)
    rl = jnp.where(lane < E, logits, NEG)
    mx = jnp.max(rl, axis=-1, keepdims=True)
    ex = jnp.exp(rl - mx)
    probs = ex / jnp.sum(ex, axis=-1, keepdims=True)
    p_ref[...] = probs
    v1 = jnp.max(probs, axis=-1, keepdims=True)
    i1 = jnp.min(jnp.where((probs == v1) & (lane < E), lane, 128),
                 axis=-1, keepdims=True)
    probs2 = jnp.where((lane == i1) | (lane >= E), NEG, probs)
    v2 = jnp.max(probs2, axis=-1, keepdims=True)
    i2 = jnp.min(jnp.where((probs2 == v2) & (lane < E), lane, 128),
                 axis=-1, keepdims=True)
    tot = v1 + v2
    gates = (jnp.where(lane == i1, v1 / tot, 0.0)
             + jnp.where(lane == i2, v2 / tot, 0.0))
    sg = jax.nn.sigmoid(logits[:, E:E + 1])
    g_ref[...] = gates + jnp.where(lane == E, sg, 0.0)


def _post_router(x, o, Wo, ln2, Wrcat, bs=512):
    nb = S // bs
    return pl.pallas_call(
        _post_router_body,
        grid=(nb,),
        in_specs=[
            pl.BlockSpec((bs, D), lambda i: (i, 0)),
            pl.BlockSpec((bs, D), lambda i: (i, 0)),
            pl.BlockSpec((D, D), lambda i: (0, 0)),
            pl.BlockSpec((1, D), lambda i: (0, 0)),
            pl.BlockSpec((D, 128), lambda i: (0, 0)),
        ],
        out_specs=[
            pl.BlockSpec((bs, D), lambda i: (i, 0)),
            pl.BlockSpec((bs, D), lambda i: (i, 0)),
            pl.BlockSpec((bs, 128), lambda i: (i, 0)),
            pl.BlockSpec((bs, 128), lambda i: (i, 0)),
        ],
        out_shape=[
            jax.ShapeDtypeStruct((S, D), jnp.float32),
            jax.ShapeDtypeStruct((S, D), jnp.bfloat16),
            jax.ShapeDtypeStruct((S, 128), jnp.float32),
            jax.ShapeDtypeStruct((S, 128), jnp.float32),
        ],
    )(x, o, Wo, ln2.reshape(1, D), Wrcat)


def _moe_body(h2_ref, w1_ref, w3_ref, w2_ref, g_ref, out_ref):
    e = pl.program_id(1)
    h2 = h2_ref[...]
    a = _dot(h2, w1_ref[0])
    bmat = _dot(h2, w3_ref[0])
    inner = (a * jax.nn.sigmoid(a)) * bmat
    ye = _dot(inner.astype(jnp.bfloat16), w2_ref[0])
    contrib = ye * g_ref[0, 0]

    @pl.when(e == 0)
    def _():
        out_ref[...] = contrib

    @pl.when(e != 0)
    def _():
        out_ref[...] += contrib


def _moe_dense(h2b, W1c, W3c, W2c, gcol, bs=512):
    nb = S // bs
    return pl.pallas_call(
        _moe_body,
        grid=(nb, E + 1),
        in_specs=[
            pl.BlockSpec((bs, D), lambda i, e: (i, 0)),
            pl.BlockSpec((1, D, F), lambda i, e: (e, 0, 0)),
            pl.BlockSpec((1, D, F), lambda i, e: (e, 0, 0)),
            pl.BlockSpec((1, F, D), lambda i, e: (e, 0, 0)),
            pl.BlockSpec((1, 1, bs, 1), lambda i, e: (e, i, 0, 0)),
        ],
        out_specs=pl.BlockSpec((bs, D), lambda i, e: (i, 0)),
        out_shape=jax.ShapeDtypeStruct((S, D), jnp.float32),
    )(h2b, W1c, W3c, W2c, gcol)


def _final_body(x2_ref, moe_ref, lnf_ref, wh_ref, bh_ref, t_ref, m_ref,
                g_ref, p_ref, acc_ref, loss_ref, *, nb):
    i = pl.program_id(0)

    @pl.when(i == 0)
    def _():
        acc_ref[...] = jnp.zeros_like(acc_ref)

    x3 = x2_ref[...] + moe_ref[...]
    hf = _rmsnorm(x3, lnf_ref[...])
    pred = _dot(hf, wh_ref[...])[:, :1] + bh_ref[...]
    diff = pred - t_ref[...]
    msk = m_ref[...]
    lane = jax.lax.broadcasted_iota(jnp.int32, g_ref.shape, 1)
    fsel = ((g_ref[...] > 0) & (lane < E)).astype(jnp.float32)
    acc_ref[0:1, 0:1] += jnp.sum(diff * diff * msk, axis=(0, 1),
                                 keepdims=True)
    acc_ref[1:2, 0:1] += jnp.sum(msk, axis=(0, 1), keepdims=True)
    acc_ref[2:3, :] += jnp.sum(fsel, axis=0, keepdims=True)
    acc_ref[3:4, :] += jnp.sum(p_ref[...], axis=0, keepdims=True)

    @pl.when(i == nb - 1)
    def _():
        mse = acc_ref[0:1, 0:1] / jnp.maximum(acc_ref[1:2, 0:1], 1.0)
        lane1 = jax.lax.broadcasted_iota(jnp.int32, (1, 128), 1)
        fp = jnp.where(lane1 < E, acc_ref[2:3, :] * acc_ref[3:4, :], 0.0)
        aux = (E / (S * S * 1.0)) * jnp.sum(fp, axis=(0, 1), keepdims=True)
        loss_ref[...] = mse + 0.02 * aux


def _final_loss(x2, moe, lnf, Whcat, b_head, t_col, m_col, gates, probs,
                bs=512):
    nb = S // bs
    body = functools.partial(_final_body, nb=nb)
    acc, loss = pl.pallas_call(
        body,
        grid=(nb,),
        in_specs=[
            pl.BlockSpec((bs, D), lambda i: (i, 0)),
            pl.BlockSpec((bs, D), lambda i: (i, 0)),
            pl.BlockSpec((1, D), lambda i: (0, 0)),
            pl.BlockSpec((D, 128), lambda i: (0, 0)),
            pl.BlockSpec((1, 1), lambda i: (0, 0)),
            pl.BlockSpec((bs, 1), lambda i: (i, 0)),
            pl.BlockSpec((bs, 1), lambda i: (i, 0)),
            pl.BlockSpec((bs, 128), lambda i: (i, 0)),
            pl.BlockSpec((bs, 128), lambda i: (i, 0)),
        ],
        out_specs=[
            pl.BlockSpec((4, 128), lambda i: (0, 0)),
            pl.BlockSpec((1, 1), lambda i: (0, 0)),
        ],
        out_shape=[
            jax.ShapeDtypeStruct((4, 128), jnp.float32),
            jax.ShapeDtypeStruct((1, 1), jnp.float32),
        ],
    )(x2, moe, lnf.reshape(1, D), Whcat, b_head.reshape(1, 1), t_col, m_col,
      gates, probs)
    return loss


def kernel(context, target, mask, W_in, b_in, ln1, ln2, lnf, Wq, Wk, Wv, Wo,
           W_router, W1, W3, W2, Ws1, Ws3, Ws2, W_sg, W_head, b_head):
    bf = jnp.bfloat16
    c_col = context.reshape(S, 1)
    x, q, k, v = _embed_qkv(c_col, W_in, b_in, ln1,
                            Wq.astype(bf), Wk.astype(bf), Wv.astype(bf))

    qh = q.reshape(S, H, DH).transpose(1, 0, 2)
    kh = k.reshape(S, H, DH).transpose(1, 0, 2)
    vh = v.reshape(S, H, DH).transpose(1, 0, 2)
    oh = _flash_attn(qh, kh, vh)
    o = oh.transpose(1, 0, 2).reshape(S, D)

    Wrcat = jnp.zeros((D, 128), jnp.float32)
    Wrcat = Wrcat.at[:, :E].set(W_router).at[:, E:E + 1].set(W_sg)
    x2, h2b, gates, probs = _post_router(x, o.astype(bf), Wo.astype(bf),
                                         ln2, Wrcat)

    W1c = jnp.concatenate([W1, Ws1[None]], axis=0).astype(bf)
    W3c = jnp.concatenate([W3, Ws3[None]], axis=0).astype(bf)
    W2c = jnp.concatenate([W2, Ws2[None]], axis=0).astype(bf)
    bs = 512
    gcol = gates[:, :E + 1].T.reshape(E + 1, S // bs, bs, 1)
    moe = _moe_dense(h2b, W1c, W3c, W2c, gcol, bs=bs)

    Whcat = jnp.zeros((D, 128), jnp.float32).at[:, :1].set(W_head)
    loss = _final_loss(x2, moe, lnf, Whcat, b_head, target.reshape(S, 1),
                       mask.reshape(S, 1), gates, probs)
    return jnp.reshape(loss, ())
